# baseline (speedup 1.0000x reference)
import functools

import jax
import jax.numpy as jnp
from jax import lax
from jax.experimental import pallas as pl
from jax.experimental.pallas import tpu as pltpu
from jax.experimental.pallas import tpu_sc as plsc

_NC = 2
_NS = 16
_NW = _NC * _NS
_LANES = 16
_CHUNK_A = 128
_UNROLL = 4


def _pooled_mean_sc(ids, emb_table):
    batch, seq = ids.shape
    hidden = emb_table.shape[1]
    chunk_b = seq - _CHUNK_A
    rows_w = batch // _NW
    n_col = hidden // _LANES
    inv_seq = jnp.float32(1.0 / seq)
    mesh = plsc.VectorSubcoreMesh(core_axis_name="c", subcore_axis_name="s")

    @functools.partial(
        pl.kernel,
        mesh=mesh,
        out_type=jax.ShapeDtypeStruct((batch, hidden), jnp.float32),
        compiler_params=pltpu.CompilerParams(use_tc_tiling_on_sc=False),
        scratch_types=[
            pltpu.VMEM((rows_w, seq), jnp.int32),
            [
                pltpu.VMEM((_CHUNK_A, hidden), jnp.float32),
                pltpu.VMEM((chunk_b, hidden), jnp.float32),
                pltpu.VMEM((_CHUNK_A, hidden), jnp.float32),
                pltpu.VMEM((chunk_b, hidden), jnp.float32),
            ],
            pltpu.VMEM((rows_w, hidden), jnp.float32),
            [pltpu.SemaphoreType.DMA for _ in range(4)],
        ],
    )
    def k(idx_hbm, table_hbm, out_hbm, idx_v, bufs, pooled_v, sems):
        wid = lax.axis_index("s") * _NC + lax.axis_index("c")
        pltpu.sync_copy(idx_hbm.at[pl.ds(wid * rows_w, rows_w)], idx_v)

        def chunk_idx(row, part):
            if part == 0:
                return idx_v.at[row, pl.ds(0, _CHUNK_A)]
            return idx_v.at[row, pl.ds(_CHUNK_A, chunk_b)]

        for r2 in range(2):
            for part in range(2):
                b = 2 * r2 + part
                pltpu.async_copy(table_hbm.at[chunk_idx(r2, part)], bufs[b], sems[b])

        def reduce_chunk(buf, n, accs):
            assert n % _UNROLL == 0

            def body(i, a):
                s = i * _UNROLL
                for u in range(_UNROLL):
                    a = tuple(
                        a[c] + buf[s + u, pl.ds(c * _LANES, _LANES)]
                        for c in range(n_col)
                    )
                return a

            return lax.fori_loop(0, n // _UNROLL, body, accs)

        def group_body(g, carry):
            for r2 in range(2):
                row = 2 * g + r2
                accs = tuple(
                    jnp.zeros((_LANES,), jnp.float32) for _ in range(n_col)
                )
                for part in range(2):
                    b = 2 * r2 + part
                    n = _CHUNK_A if part == 0 else chunk_b
                    pltpu.make_async_copy(
                        table_hbm.at[chunk_idx(row, part)], bufs[b], sems[b]
                    ).wait()
                    accs = reduce_chunk(bufs[b], n, accs)

                    @pl.when(row + 2 < rows_w)
                    def _():
                        pltpu.async_copy(
                            table_hbm.at[chunk_idx(row + 2, part)],
                            bufs[b],
                            sems[b],
                        )

                for c in range(n_col):
                    pooled_v[row, pl.ds(c * _LANES, _LANES)] = accs[c] * inv_seq
            return carry

        lax.fori_loop(0, rows_w // 2, group_body, 0)
        pltpu.sync_copy(pooled_v, out_hbm.at[pl.ds(wid * rows_w, rows_w)])

    return k(ids, emb_table)


def _mlp_tc(pooled, W1, b1, W2, b2):
    def body(p_ref, w1_ref, b1_ref, w2_ref, b2_ref, o_ref):
        h = jnp.dot(p_ref[...], w1_ref[...], preferred_element_type=jnp.float32)
        h = jnp.maximum(h + b1_ref[...], 0.0)
        o_ref[...] = (
            jnp.dot(h, w2_ref[...], preferred_element_type=jnp.float32)
            + b2_ref[...]
        )

    return pl.pallas_call(
        body,
        out_shape=jax.ShapeDtypeStruct((pooled.shape[0], 1), jnp.float32),
    )(pooled, W1, b1, W2, b2)


def kernel(input_ids, emb_table, W1, b1, W2, b2):
    batch = input_ids.shape[0]
    pooled = _pooled_mean_sc(input_ids.astype(jnp.int32), emb_table)
    out = _mlp_tc(
        pooled,
        W1,
        b1.reshape(1, -1).astype(jnp.float32),
        W2,
        b2.reshape(1, 1).astype(jnp.float32),
    )
    return out.reshape(batch)


# speedup vs baseline: 1.0004x; 1.0004x over previous
import functools

import jax
import jax.numpy as jnp
from jax import lax
from jax.experimental import pallas as pl
from jax.experimental.pallas import tpu as pltpu
from jax.experimental.pallas import tpu_sc as plsc

_NC = 2
_NS = 16
_NW = _NC * _NS
_LANES = 16
_CHUNK_A = 128
_UNROLL = 4


def _pooled_mean_sc(ids2, emb_table, batch, seq):
    hidden = emb_table.shape[1]
    chunk_b = seq - _CHUNK_A
    rows_w = batch // _NW
    flat_w = rows_w * seq
    idxrows_w = flat_w // 128
    n_col = hidden // _LANES
    inv_seq = jnp.float32(1.0 / seq)
    mesh = plsc.VectorSubcoreMesh(core_axis_name="c", subcore_axis_name="s")

    @functools.partial(
        pl.kernel,
        mesh=mesh,
        out_type=jax.ShapeDtypeStruct((batch, hidden), jnp.float32),
        compiler_params=pltpu.CompilerParams(use_tc_tiling_on_sc=False),
        scratch_types=[
            pltpu.VMEM((flat_w,), jnp.int32),
            [
                pltpu.VMEM((_CHUNK_A, hidden), jnp.float32),
                pltpu.VMEM((chunk_b, hidden), jnp.float32),
                pltpu.VMEM((_CHUNK_A, hidden), jnp.float32),
                pltpu.VMEM((chunk_b, hidden), jnp.float32),
            ],
            pltpu.VMEM((rows_w, hidden), jnp.float32),
            [pltpu.SemaphoreType.DMA for _ in range(4)],
            pltpu.SemaphoreType.DMA,
        ],
    )
    def k(idx_hbm, table_hbm, out_hbm, idx_v, bufs, pooled_v, sems, isem):
        wid = lax.axis_index("s") * _NC + lax.axis_index("c")
        base = wid * idxrows_w

        def stage(j, carry):
            pltpu.async_copy(
                idx_hbm.at[base + j], idx_v.at[pl.ds(j * 128, 128)], isem
            )
            return carry

        lax.fori_loop(0, idxrows_w, stage, 0)

        def drain(j, carry):
            pltpu.make_async_copy(
                idx_hbm.at[base + j], idx_v.at[pl.ds(j * 128, 128)], isem
            ).wait()
            return carry

        lax.fori_loop(0, idxrows_w, drain, 0)

        def chunk_idx(row, part):
            if part == 0:
                return idx_v.at[pl.ds(row * seq, _CHUNK_A)]
            return idx_v.at[pl.ds(row * seq + _CHUNK_A, chunk_b)]

        for r2 in range(2):
            for part in range(2):
                b = 2 * r2 + part
                pltpu.async_copy(table_hbm.at[chunk_idx(r2, part)], bufs[b], sems[b])

        def reduce_chunk(buf, n, accs):
            assert n % _UNROLL == 0

            def body(i, a):
                s = i * _UNROLL
                for u in range(_UNROLL):
                    a = tuple(
                        a[c] + buf[s + u, pl.ds(c * _LANES, _LANES)]
                        for c in range(n_col)
                    )
                return a

            return lax.fori_loop(0, n // _UNROLL, body, accs)

        def group_body(g, carry):
            for r2 in range(2):
                row = 2 * g + r2
                accs = tuple(
                    jnp.zeros((_LANES,), jnp.float32) for _ in range(n_col)
                )
                for part in range(2):
                    b = 2 * r2 + part
                    n = _CHUNK_A if part == 0 else chunk_b
                    pltpu.make_async_copy(
                        table_hbm.at[chunk_idx(row, part)], bufs[b], sems[b]
                    ).wait()
                    accs = reduce_chunk(bufs[b], n, accs)

                    @pl.when(row + 2 < rows_w)
                    def _():
                        pltpu.async_copy(
                            table_hbm.at[chunk_idx(row + 2, part)],
                            bufs[b],
                            sems[b],
                        )

                for c in range(n_col):
                    pooled_v[row, pl.ds(c * _LANES, _LANES)] = accs[c] * inv_seq
            return carry

        lax.fori_loop(0, rows_w // 2, group_body, 0)
        pltpu.sync_copy(pooled_v, out_hbm.at[pl.ds(wid * rows_w, rows_w)])

    return k(ids2, emb_table)


def _mlp_tc(pooled, W1, b1, W2, b2):
    def body(p_ref, w1_ref, b1_ref, w2_ref, b2_ref, o_ref):
        h = jnp.dot(p_ref[...], w1_ref[...], preferred_element_type=jnp.float32)
        h = jnp.maximum(h + b1_ref[...], 0.0)
        o_ref[...] = (
            jnp.dot(h, w2_ref[...], preferred_element_type=jnp.float32)
            + b2_ref[...]
        )

    return pl.pallas_call(
        body,
        out_shape=jax.ShapeDtypeStruct((pooled.shape[0], 1), jnp.float32),
    )(pooled, W1, b1, W2, b2)


def kernel(input_ids, emb_table, W1, b1, W2, b2):
    batch, seq = input_ids.shape
    ids2 = input_ids.astype(jnp.int32).reshape(batch * seq // 128, 128)
    pooled = _pooled_mean_sc(ids2, emb_table, batch, seq)
    out = _mlp_tc(
        pooled,
        W1,
        b1.reshape(1, -1).astype(jnp.float32),
        W2,
        b2.reshape(1, 1).astype(jnp.float32),
    )
    return out.reshape(batch)


# speedup vs baseline: 1.0016x; 1.0011x over previous
import functools

import jax
import jax.numpy as jnp
from jax import lax
from jax.experimental import pallas as pl
from jax.experimental.pallas import tpu as pltpu
from jax.experimental.pallas import tpu_sc as plsc

_NC = 2
_NS = 16
_NW = _NC * _NS
_LANES = 16
_CHUNK_A = 128
_UNROLL = 4


def _flatten_ids_sc(ids):
    batch, seq = ids.shape
    rows_w = batch // _NW
    slabs_w = rows_w // 8
    chunk_b = seq - _CHUNK_A
    mesh = plsc.VectorSubcoreMesh(core_axis_name="c", subcore_axis_name="s")

    @functools.partial(
        pl.kernel,
        mesh=mesh,
        out_type=jax.ShapeDtypeStruct((batch * seq,), jnp.int32),
        compiler_params=pltpu.CompilerParams(use_tc_tiling_on_sc=True),
        scratch_types=[
            [pltpu.VMEM((8, _CHUNK_A), jnp.int32) for _ in range(2)],
            [pltpu.VMEM((8, chunk_b), jnp.int32) for _ in range(2)],
            [pltpu.VMEM((8 * seq,), jnp.int32) for _ in range(2)],
            [pltpu.SemaphoreType.DMA for _ in range(2)],
            [pltpu.SemaphoreType.DMA for _ in range(2)],
        ],
    )
    def k(ids_hbm, out_hbm, vas, vbs, vcs, sems_in, sems_out):
        wid = lax.axis_index("s") * _NC + lax.axis_index("c")
        row0 = wid * rows_w

        def fire_in(j, p):
            pltpu.async_copy(
                ids_hbm.at[pl.ds(row0 + 8 * j, 8), pl.ds(0, _CHUNK_A)],
                vas[p],
                sems_in[p],
            )
            pltpu.async_copy(
                ids_hbm.at[pl.ds(row0 + 8 * j, 8), pl.ds(_CHUNK_A, chunk_b)],
                vbs[p],
                sems_in[p],
            )

        def wait_in(j, p):
            pltpu.make_async_copy(
                ids_hbm.at[pl.ds(row0 + 8 * j, 8), pl.ds(0, _CHUNK_A)],
                vas[p],
                sems_in[p],
            ).wait()
            pltpu.make_async_copy(
                ids_hbm.at[pl.ds(row0 + 8 * j, 8), pl.ds(_CHUNK_A, chunk_b)],
                vbs[p],
                sems_in[p],
            ).wait()

        def out_desc(j, p):
            return pltpu.make_async_copy(
                vcs[p], out_hbm.at[pl.ds((row0 + 8 * j) * seq, 8 * seq)],
                sems_out[p],
            )

        for p in range(2):
            fire_in(p, p)

        def slab_body(g, carry):
            for p in range(2):
                j = 2 * g + p
                wait_in(j, p)

                @pl.when(g > 0)
                def _():
                    out_desc(j, p).wait()

                for r in range(8):
                    for c in range(_CHUNK_A // _LANES):
                        vcs[p][pl.ds(seq * r + _LANES * c, _LANES)] = vas[p][
                            r, pl.ds(_LANES * c, _LANES)
                        ]
                    nb_full = chunk_b // _LANES
                    for c in range(nb_full):
                        vcs[p][pl.ds(seq * r + _CHUNK_A + _LANES * c, _LANES)] = (
                            vbs[p][r, pl.ds(_LANES * c, _LANES)]
                        )
                    if chunk_b % _LANES:
                        off = chunk_b - _LANES
                        vcs[p][pl.ds(seq * r + _CHUNK_A + off, _LANES)] = vbs[p][
                            r, pl.ds(off, _LANES)
                        ]
                out_desc(j, p).start()

                @pl.when(j + 2 < slabs_w)
                def _():
                    fire_in(j + 2, p)

            return carry

        lax.fori_loop(0, slabs_w // 2, slab_body, 0)
        for p in range(2):
            out_desc(slabs_w - 2 + p, p).wait()

    return k(ids)


def _pooled_mean_sc(ids1d, emb_table, batch, seq):
    hidden = emb_table.shape[1]
    chunk_b = seq - _CHUNK_A
    rows_w = batch // _NW
    flat_w = rows_w * seq
    idxrows_w = flat_w // 128
    n_col = hidden // _LANES
    inv_seq = jnp.float32(1.0 / seq)
    mesh = plsc.VectorSubcoreMesh(core_axis_name="c", subcore_axis_name="s")

    @functools.partial(
        pl.kernel,
        mesh=mesh,
        out_type=jax.ShapeDtypeStruct((batch, hidden), jnp.float32),
        compiler_params=pltpu.CompilerParams(use_tc_tiling_on_sc=False),
        scratch_types=[
            pltpu.VMEM((flat_w,), jnp.int32),
            [
                pltpu.VMEM((_CHUNK_A, hidden), jnp.float32),
                pltpu.VMEM((chunk_b, hidden), jnp.float32),
                pltpu.VMEM((_CHUNK_A, hidden), jnp.float32),
                pltpu.VMEM((chunk_b, hidden), jnp.float32),
            ],
            pltpu.VMEM((rows_w, hidden), jnp.float32),
            [pltpu.SemaphoreType.DMA for _ in range(4)],
            pltpu.SemaphoreType.DMA,
        ],
    )
    def k(idx_hbm, table_hbm, out_hbm, idx_v, bufs, pooled_v, sems, isem):
        wid = lax.axis_index("s") * _NC + lax.axis_index("c")
        pltpu.sync_copy(idx_hbm.at[pl.ds(wid * flat_w, flat_w)], idx_v)

        def chunk_idx(row, part):
            if part == 0:
                return idx_v.at[pl.ds(row * seq, _CHUNK_A)]
            return idx_v.at[pl.ds(row * seq + _CHUNK_A, chunk_b)]

        for r2 in range(2):
            for part in range(2):
                b = 2 * r2 + part
                pltpu.async_copy(table_hbm.at[chunk_idx(r2, part)], bufs[b], sems[b])

        def reduce_chunk(buf, n, accs):
            assert n % _UNROLL == 0

            def body(i, a):
                s = i * _UNROLL
                for u in range(_UNROLL):
                    a = tuple(
                        a[c] + buf[s + u, pl.ds(c * _LANES, _LANES)]
                        for c in range(n_col)
                    )
                return a

            return lax.fori_loop(0, n // _UNROLL, body, accs)

        def group_body(g, carry):
            for r2 in range(2):
                row = 2 * g + r2
                accs = tuple(
                    jnp.zeros((_LANES,), jnp.float32) for _ in range(n_col)
                )
                for part in range(2):
                    b = 2 * r2 + part
                    n = _CHUNK_A if part == 0 else chunk_b
                    pltpu.make_async_copy(
                        table_hbm.at[chunk_idx(row, part)], bufs[b], sems[b]
                    ).wait()
                    accs = reduce_chunk(bufs[b], n, accs)

                    @pl.when(row + 2 < rows_w)
                    def _():
                        pltpu.async_copy(
                            table_hbm.at[chunk_idx(row + 2, part)],
                            bufs[b],
                            sems[b],
                        )

                for c in range(n_col):
                    pooled_v[row, pl.ds(c * _LANES, _LANES)] = accs[c] * inv_seq
            return carry

        lax.fori_loop(0, rows_w // 2, group_body, 0)
        pltpu.sync_copy(pooled_v, out_hbm.at[pl.ds(wid * rows_w, rows_w)])

    return k(ids1d, emb_table)


def _mlp_tc(pooled, W1, b1, W2, b2):
    def body(p_ref, w1_ref, b1_ref, w2_ref, b2_ref, o_ref):
        h = jnp.dot(p_ref[...], w1_ref[...], preferred_element_type=jnp.float32)
        h = jnp.maximum(h + b1_ref[...], 0.0)
        o_ref[...] = (
            jnp.dot(h, w2_ref[...], preferred_element_type=jnp.float32)
            + b2_ref[...]
        )

    return pl.pallas_call(
        body,
        out_shape=jax.ShapeDtypeStruct((pooled.shape[0], 1), jnp.float32),
    )(pooled, W1, b1, W2, b2)


def kernel(input_ids, emb_table, W1, b1, W2, b2):
    batch, seq = input_ids.shape
    ids1d = _flatten_ids_sc(input_ids.astype(jnp.int32))
    pooled = _pooled_mean_sc(ids1d, emb_table, batch, seq)
    out = _mlp_tc(
        pooled,
        W1,
        b1.reshape(1, -1).astype(jnp.float32),
        W2,
        b2.reshape(1, 1).astype(jnp.float32),
    )
    return out.reshape(batch)
